# Initial kernel scaffold; baseline (speedup 1.0000x reference)
import jax
import jax.numpy as jnp
from jax.experimental import pallas as pl


def kernel(x, gate_w, fc1_w, fc2_w, sfc1, sfc2, sfc3):
    raise NotImplementedError("write your pallas kernel here")


# speedup vs baseline: 1.8825x; 1.8825x over previous
import functools

import jax
import jax.numpy as jnp
from jax.experimental import pallas as pl
from jax.experimental.pallas import tpu as pltpu
from jax.experimental.pallas import tpu_sc as plsc

D = 768
E = 8
HID = 256
NTOK = 2048
T4 = 256
MAXT = NTOK * 2 // T4 + 7
NS = MAXT * T4


def _silu(v):
    return v * jax.nn.sigmoid(v)


def _meta_body(x_ref, gw_ref, s1_ref, s2_ref, s3_ref,
               po_ref, te_ref, sh_ref, counts_ref):
    x = x_ref[...]
    gw = gw_ref[...]
    tr = (((1,), (1,)), ((), ()))
    a = jax.lax.dot_general(x, s1_ref[...], tr,
                            preferred_element_type=jnp.float32)
    b = jax.lax.dot_general(x, s2_ref[...], tr,
                            preferred_element_type=jnp.float32)
    sh_ref[...] = jax.lax.dot_general(_silu(a) * b, s3_ref[...], tr,
                                      preferred_element_type=jnp.float32)
    gt = jax.lax.dot_general(gw, x, (((1,), (1,)), ((), ())),
                             preferred_element_type=jnp.float32)
    e8 = jax.lax.broadcasted_iota(jnp.int32, gt.shape, 0)
    m1 = jnp.max(gt, axis=0, keepdims=True)
    i1 = jnp.min(jnp.where(gt == m1, e8, E), axis=0, keepdims=True)
    g2 = jnp.where(e8 == i1, -jnp.inf, gt)
    m2 = jnp.max(g2, axis=0, keepdims=True)
    i2 = jnp.min(jnp.where(g2 == m2, e8, E), axis=0, keepdims=True)
    d = jnp.exp(m2 - m1)
    p1 = 1.0 / (1.0 + d)
    p2 = 1.0 - p1
    w = (jnp.where(e8 == i1, p1, 0.0)
         + jnp.where(e8 == i2, p2, 0.0))
    pos_m = w > 0.0
    eA = jnp.min(jnp.where(pos_m, e8, E), axis=0, keepdims=True)
    eB = jnp.max(jnp.where(pos_m, e8, -1), axis=0, keepdims=True)
    ma = (e8 == eA).astype(jnp.float32)
    mb = (e8 == eB).astype(jnp.float32)

    nch = NTOK // 128
    for c in range(nch):
        sl = (slice(None), slice(c * 128, (c + 1) * 128))
        counts_ref[:, c:c + 1] = jnp.sum(ma[sl], axis=1, keepdims=True)
        counts_ref[:, nch + c:nch + c + 1] = jnp.sum(mb[sl], axis=1,
                                                     keepdims=True)
    counts = counts_ref[...]

    r32 = jax.lax.broadcasted_iota(jnp.int32, (2 * nch, 2 * nch), 0)
    c32 = jax.lax.broadcasted_iota(jnp.int32, (2 * nch, 2 * nch), 1)
    triu32 = (r32 < c32).astype(jnp.float32)
    prefix = jnp.dot(counts, triu32,
                     preferred_element_type=jnp.float32)

    tot = jnp.sum(counts, axis=1, keepdims=True)
    pci = ((tot.astype(jnp.int32) + T4 - 1) // T4) * T4
    pcf = pci.astype(jnp.float32)
    r8 = jax.lax.broadcasted_iota(jnp.int32, (E, E), 0)
    c8 = jax.lax.broadcasted_iota(jnp.int32, (E, E), 1)
    tril8s = (c8 < r8).astype(jnp.float32)
    po = jnp.dot(tril8s, pcf, preferred_element_type=jnp.float32)

    m_row = jax.lax.broadcasted_iota(
        jnp.int32, (E, 32), 1).astype(jnp.float32) * T4
    cond = (m_row >= po) & (m_row < po + pcf)
    e_col = jax.lax.broadcasted_iota(jnp.int32, (E, 32), 0).astype(jnp.float32)
    te = jnp.sum(jnp.where(cond, e_col, 0.0), axis=0, keepdims=True)
    te_ref[...] = jnp.broadcast_to(te, (E, 32))

    rr = jax.lax.broadcasted_iota(jnp.int32, (128, 128), 0)
    cc = jax.lax.broadcasted_iota(jnp.int32, (128, 128), 1)
    triu128 = (rr < cc).astype(jnp.float32)
    row4 = jax.lax.broadcasted_iota(jnp.int32, (E, 128), 0)
    for c in range(nch):
        sl = (slice(None), slice(c * 128, (c + 1) * 128))
        wc = w[sl]
        pos_ab = []
        for mx, chunk_idx in ((ma[sl], c), (mb[sl], nch + c)):
            r = jnp.dot(mx, triu128, preferred_element_type=jnp.float32)
            rank = jnp.sum(r * mx, axis=0, keepdims=True)
            posel = jnp.sum(mx * po, axis=0, keepdims=True)
            prefsel = jnp.sum(mx * prefix[:, chunk_idx:chunk_idx + 1],
                              axis=0, keepdims=True)
            pos_ab.append(rank + posel + prefsel)
        pa = jnp.sum(ma[sl] * wc, axis=0, keepdims=True)
        pb = jnp.sum(wc, axis=0, keepdims=True) - pa
        blk = (jnp.where(row4 == 0, pos_ab[0], 0.0)
               + jnp.where(row4 == 1, pos_ab[1], 0.0)
               + jnp.where(row4 == 2, pa, 0.0)
               + jnp.where(row4 == 3, pb, 0.0))
        po_ref[:, c * 128:(c + 1) * 128] = blk


def _metadata(xt, gw, s1, s2, s3):
    s = s1.shape[0]
    return pl.pallas_call(
        _meta_body,
        grid=(1,),
        in_specs=[pl.BlockSpec((NTOK, D), lambda i: (0, 0)),
                  pl.BlockSpec((E, D), lambda i: (0, 0)),
                  pl.BlockSpec((s, D), lambda i: (0, 0)),
                  pl.BlockSpec((s, D), lambda i: (0, 0)),
                  pl.BlockSpec((D, s), lambda i: (0, 0))],
        out_specs=[pl.BlockSpec((E, NTOK), lambda i: (0, 0)),
                   pl.BlockSpec((E, 32), lambda i: (0, 0)),
                   pl.BlockSpec((NTOK, D), lambda i: (0, 0))],
        out_shape=[jax.ShapeDtypeStruct((E, NTOK), jnp.float32),
                   jax.ShapeDtypeStruct((E, 32), jnp.float32),
                   jax.ShapeDtypeStruct((NTOK, D), jnp.float32)],
        scratch_shapes=[pltpu.VMEM((E, 32), jnp.float32)],
    )(xt, gw, s1, s2, s3)


def _shared_body(x_ref, s1_ref, s2_ref, s3_ref, o_ref):
    x = x_ref[...]
    a = jnp.dot(x, s1_ref[...], preferred_element_type=jnp.float32)
    b = jnp.dot(x, s2_ref[...], preferred_element_type=jnp.float32)
    o_ref[...] = jnp.dot(_silu(a) * b, s3_ref[...],
                         preferred_element_type=jnp.float32)


def _shared_expert(xt, s1_t, s2_t, s3_t, tile=256):
    s = s1_t.shape[1]
    return pl.pallas_call(
        _shared_body,
        grid=(NTOK // tile,),
        in_specs=[pl.BlockSpec((tile, D), lambda i: (i, 0)),
                  pl.BlockSpec((D, s), lambda i: (0, 0)),
                  pl.BlockSpec((D, s), lambda i: (0, 0)),
                  pl.BlockSpec((s, D), lambda i: (0, 0))],
        out_specs=pl.BlockSpec((tile, D), lambda i: (i, 0)),
        out_shape=jax.ShapeDtypeStruct((NTOK, D), jnp.float32),
    )(xt, s1_t, s2_t, s3_t)


def _ffn_body(te_ref, x_ref, w1_ref, w2_ref, o_ref):
    x = x_ref[...]
    tr = (((1,), (1,)), ((), ()))
    h = _silu(jax.lax.dot_general(x, w1_ref[0], tr,
                                  preferred_element_type=jnp.float32))
    o_ref[...] = jax.lax.dot_general(h, w2_ref[0], tr,
                                     preferred_element_type=jnp.float32)


def _grouped_ffn(te, xs, fc1_w, fc2_w):
    grid_spec = pltpu.PrefetchScalarGridSpec(
        num_scalar_prefetch=1,
        grid=(MAXT,),
        in_specs=[
            pl.BlockSpec((T4, D), lambda i, te: (i, 0)),
            pl.BlockSpec((1, HID, D), lambda i, te: (te[i], 0, 0)),
            pl.BlockSpec((1, D, HID), lambda i, te: (te[i], 0, 0)),
        ],
        out_specs=pl.BlockSpec((T4, D), lambda i, te: (i, 0)),
    )
    return pl.pallas_call(
        _ffn_body,
        grid_spec=grid_spec,
        out_shape=jax.ShapeDtypeStruct((NS, D), jnp.float32),
    )(te, xs, fc1_w, fc2_w)


_NW = 32
_TPW = NTOK // _NW


def _sc_scatter_x(xt, idx0, idx1):
    mesh = plsc.VectorSubcoreMesh(core_axis_name="core",
                                  subcore_axis_name="subcore")

    @functools.partial(
        pl.kernel, mesh=mesh,
        out_type=jax.ShapeDtypeStruct((NS, D), jnp.float32),
        scratch_types=[pltpu.VMEM((_TPW, D), jnp.float32),
                       pltpu.VMEM((_TPW,), jnp.int32),
                       pltpu.VMEM((_TPW,), jnp.int32),
                       pltpu.SemaphoreType.DMA])
    def k(x_hbm, i0_hbm, i1_hbm, xs_hbm, rows_v, idx0_v, idx1_v, sem):
        wid = (jax.lax.axis_index("subcore") * 2
               + jax.lax.axis_index("core"))
        base = wid * _TPW
        pltpu.sync_copy(x_hbm.at[pl.ds(base, _TPW)], rows_v)
        pltpu.sync_copy(i0_hbm.at[pl.ds(base, _TPW)], idx0_v)
        pltpu.sync_copy(i1_hbm.at[pl.ds(base, _TPW)], idx1_v)
        pltpu.async_copy(rows_v, xs_hbm.at[idx0_v], sem).wait()
        pltpu.async_copy(rows_v, xs_hbm.at[idx1_v], sem).wait()

    return k(xt, idx0, idx1)


def _sc_combine(ys, shared, idx0, idx1, p0f, p1f):
    mesh = plsc.VectorSubcoreMesh(core_axis_name="core",
                                  subcore_axis_name="subcore")
    bt = 32

    @functools.partial(
        pl.kernel, mesh=mesh,
        out_type=jax.ShapeDtypeStruct((NTOK, D), jnp.float32),
        scratch_types=[pltpu.VMEM((bt, D), jnp.float32),
                       pltpu.VMEM((bt, D), jnp.float32),
                       pltpu.VMEM((bt, D), jnp.float32),
                       pltpu.VMEM((_TPW,), jnp.int32),
                       pltpu.VMEM((_TPW,), jnp.int32),
                       pltpu.VMEM((_TPW * 16,), jnp.float32),
                       pltpu.VMEM((_TPW * 16,), jnp.float32),
                       pltpu.SemaphoreType.DMA])
    def k(ys_hbm, sh_hbm, i0_hbm, i1_hbm, p0_hbm, p1_hbm, out_hbm,
          acc_v, y0_v, y1_v, idx0_v, idx1_v, p0_v, p1_v, sem):
        wid = (jax.lax.axis_index("subcore") * 2
               + jax.lax.axis_index("core"))
        base_w = wid * _TPW
        pltpu.sync_copy(i0_hbm.at[pl.ds(base_w, _TPW)], idx0_v)
        pltpu.sync_copy(i1_hbm.at[pl.ds(base_w, _TPW)], idx1_v)
        c_p0 = pltpu.async_copy(p0_hbm.at[pl.ds(base_w * 16, _TPW * 16)],
                                p0_v, sem)
        c_p1 = pltpu.async_copy(p1_hbm.at[pl.ds(base_w * 16, _TPW * 16)],
                                p1_v, sem)
        c_p0.wait()
        c_p1.wait()

        @pl.loop(0, _TPW // bt)
        def _(bi):
            base = base_w + bi * bt
            c_sh = pltpu.async_copy(sh_hbm.at[pl.ds(base, bt)], acc_v, sem)
            c_y0 = pltpu.async_copy(
                ys_hbm.at[idx0_v.at[pl.ds(bi * bt, bt)]], y0_v, sem)
            c_y1 = pltpu.async_copy(
                ys_hbm.at[idx1_v.at[pl.ds(bi * bt, bt)]], y1_v, sem)
            c_sh.wait()
            c_y0.wait()
            c_y1.wait()

            @pl.loop(0, bt)
            def _(j):
                a = p0_v[pl.ds(bi * bt * 16 + j * 16, 16)]
                b = p1_v[pl.ds(bi * bt * 16 + j * 16, 16)]
                for v in range(D // 16):
                    sl = pl.ds(v * 16, 16)
                    acc_v[j, sl] = (acc_v[j, sl] + a * y0_v[j, sl]
                                    + b * y1_v[j, sl])

            pltpu.sync_copy(acc_v, out_hbm.at[pl.ds(base, bt)])

    return k(ys, shared, idx0, idx1, p0f, p1f)


@jax.jit
def kernel(x, gate_w, fc1_w, fc2_w, sfc1, sfc2, sfc3):
    b, l, d = x.shape
    xt = x.reshape(b * l, d)

    po, te_f, shared = _metadata(xt, gate_w, sfc1, sfc2, sfc3)
    posA = po[0].astype(jnp.int32)
    posB = po[1].astype(jnp.int32)
    pa_flat = jnp.broadcast_to(po[2][:, None], (NTOK, 16)).reshape(NTOK * 16)
    pb_flat = jnp.broadcast_to(po[3][:, None], (NTOK, 16)).reshape(NTOK * 16)
    te = te_f[0, :MAXT].astype(jnp.int32)

    xs = _sc_scatter_x(xt, posA, posB)
    ys = _grouped_ffn(te, xs, fc1_w, fc2_w)
    out = _sc_combine(ys, shared, posA, posB, pa_flat, pb_flat)
    return out.reshape(b, l, d)


# speedup vs baseline: 2.3974x; 1.2735x over previous
import functools

import jax
import jax.numpy as jnp
from jax.experimental import pallas as pl

_D = 768
_H = 256
_E = 8
_TILE = 256


def _silu(v):
    return v * jax.nn.sigmoid(v)


def _moe_body(x_ref, gw_ref, w1_ref, w2_ref, s1_ref, s2_ref, s3_ref, out_ref):
    x = x_ref[...]
    t = x.shape[0]

    g = jnp.dot(x, gw_ref[...], preferred_element_type=jnp.float32)
    e_ids = jax.lax.broadcasted_iota(jnp.int32, g.shape, 1)
    m1 = jnp.max(g, axis=1, keepdims=True)
    i1 = jnp.min(jnp.where(g == m1, e_ids, _E), axis=1, keepdims=True)
    g2 = jnp.where(e_ids == i1, -jnp.inf, g)
    m2 = jnp.max(g2, axis=1, keepdims=True)
    i2 = jnp.min(jnp.where(g2 == m2, e_ids, _E), axis=1, keepdims=True)
    d = jnp.exp(m2 - m1)
    p1 = 1.0 / (1.0 + d)
    p2 = 1.0 - p1

    h = jnp.dot(x, w1_ref[...], preferred_element_type=jnp.float32)
    ecol = jax.lax.broadcasted_iota(jnp.int32, (t, _E * _H), 1) // _H
    wfull = (jnp.where(ecol == i1, p1, 0.0)
             + jnp.where(ecol == i2, p2, 0.0))
    hw = _silu(h) * wfull
    out = jnp.dot(hw, w2_ref[...], preferred_element_type=jnp.float32)

    a = jnp.dot(x, s1_ref[...], preferred_element_type=jnp.float32)
    b = jnp.dot(x, s2_ref[...], preferred_element_type=jnp.float32)
    out = out + jnp.dot(_silu(a) * b, s3_ref[...],
                        preferred_element_type=jnp.float32)
    out_ref[...] = out


@jax.jit
def kernel(x, gate_w, fc1_w, fc2_w, sfc1, sfc2, sfc3):
    b, l, d = x.shape
    e, h, _ = fc1_w.shape
    s = sfc1.shape[0]
    n_tok = b * l
    xt = x.reshape(n_tok, d)

    gw_t = gate_w.T
    w1_t = fc1_w.reshape(e * h, d).T
    w2 = fc2_w.transpose(0, 2, 1).reshape(e * h, d)
    s1_t = sfc1.T
    s2_t = sfc2.T
    s3_t = sfc3.T

    tile = _TILE
    grid = (n_tok // tile,)

    def const(shape):
        return pl.BlockSpec(shape, lambda i: (0, 0))

    out = pl.pallas_call(
        _moe_body,
        grid=grid,
        in_specs=[
            pl.BlockSpec((tile, d), lambda i: (i, 0)),
            const(gw_t.shape), const(w1_t.shape), const(w2.shape),
            const(s1_t.shape), const(s2_t.shape), const(s3_t.shape),
        ],
        out_specs=pl.BlockSpec((tile, d), lambda i: (i, 0)),
        out_shape=jax.ShapeDtypeStruct((n_tok, d), jnp.float32),
    )(xt, gw_t, w1_t, w2, s1_t, s2_t, s3_t)
    return out.reshape(b, l, d)


# speedup vs baseline: 2.4164x; 1.0079x over previous
import functools

import jax
import jax.numpy as jnp
from jax.experimental import pallas as pl

_D = 768
_H = 256
_E = 8
_TILE = 256


def _silu(v):
    return v * jax.nn.sigmoid(v)


def _moe_body(x_ref, gw_ref, w1_ref, w2_ref, s1_ref, s2_ref, s3_ref, out_ref):
    x = x_ref[...]
    xb = x.astype(jnp.bfloat16)
    t = x.shape[0]

    g = jnp.dot(x, gw_ref[...], preferred_element_type=jnp.float32)
    e_ids = jax.lax.broadcasted_iota(jnp.int32, g.shape, 1)
    m1 = jnp.max(g, axis=1, keepdims=True)
    i1 = jnp.min(jnp.where(g == m1, e_ids, _E), axis=1, keepdims=True)
    g2 = jnp.where(e_ids == i1, -jnp.inf, g)
    m2 = jnp.max(g2, axis=1, keepdims=True)
    i2 = jnp.min(jnp.where(g2 == m2, e_ids, _E), axis=1, keepdims=True)
    d = jnp.exp(m2 - m1)
    p1 = 1.0 / (1.0 + d)
    p2 = 1.0 - p1

    h = jnp.dot(xb, w1_ref[...].astype(jnp.bfloat16),
                preferred_element_type=jnp.float32)
    ecol = jax.lax.broadcasted_iota(jnp.int32, (t, _E * _H), 1) // _H
    wfull = (jnp.where(ecol == i1, p1, 0.0)
             + jnp.where(ecol == i2, p2, 0.0))
    hw = (_silu(h) * wfull).astype(jnp.bfloat16)
    out = jnp.dot(hw, w2_ref[...].astype(jnp.bfloat16),
                  preferred_element_type=jnp.float32)

    a = jnp.dot(xb, s1_ref[...].astype(jnp.bfloat16),
                preferred_element_type=jnp.float32)
    b = jnp.dot(xb, s2_ref[...].astype(jnp.bfloat16),
                preferred_element_type=jnp.float32)
    out = out + jnp.dot((_silu(a) * b).astype(jnp.bfloat16),
                        s3_ref[...].astype(jnp.bfloat16),
                        preferred_element_type=jnp.float32)
    out_ref[...] = out


@jax.jit
def kernel(x, gate_w, fc1_w, fc2_w, sfc1, sfc2, sfc3):
    b, l, d = x.shape
    e, h, _ = fc1_w.shape
    s = sfc1.shape[0]
    n_tok = b * l
    xt = x.reshape(n_tok, d)

    gw_t = gate_w.T
    w1_t = fc1_w.reshape(e * h, d).T
    w2 = fc2_w.transpose(0, 2, 1).reshape(e * h, d)
    s1_t = sfc1.T
    s2_t = sfc2.T
    s3_t = sfc3.T

    tile = _TILE
    grid = (n_tok // tile,)

    def const(shape):
        return pl.BlockSpec(shape, lambda i: (0, 0))

    out = pl.pallas_call(
        _moe_body,
        grid=grid,
        in_specs=[
            pl.BlockSpec((tile, d), lambda i: (i, 0)),
            const(gw_t.shape), const(w1_t.shape), const(w2.shape),
            const(s1_t.shape), const(s2_t.shape), const(s3_t.shape),
        ],
        out_specs=pl.BlockSpec((tile, d), lambda i: (i, 0)),
        out_shape=jax.ShapeDtypeStruct((n_tok, d), jnp.float32),
    )(xt, gw_t, w1_t, w2, s1_t, s2_t, s3_t)
    return out.reshape(b, l, d)


# speedup vs baseline: 2.4852x; 1.0285x over previous
import functools

import jax
import jax.numpy as jnp
from jax.experimental import pallas as pl

_D = 768
_H = 256
_E = 8
_TILE = 512


def _silu(v):
    return v * jax.nn.sigmoid(v)


def _moe_body(x_ref, gw_ref, w1_ref, w2_ref, s1_ref, s2_ref, s3_ref, out_ref):
    x = x_ref[...]
    xb = x.astype(jnp.bfloat16)
    t = x.shape[0]

    g = jnp.dot(x, gw_ref[...], preferred_element_type=jnp.float32)
    e_ids = jax.lax.broadcasted_iota(jnp.int32, g.shape, 1)
    m1 = jnp.max(g, axis=1, keepdims=True)
    i1 = jnp.min(jnp.where(g == m1, e_ids, _E), axis=1, keepdims=True)
    g2 = jnp.where(e_ids == i1, -jnp.inf, g)
    m2 = jnp.max(g2, axis=1, keepdims=True)
    i2 = jnp.min(jnp.where(g2 == m2, e_ids, _E), axis=1, keepdims=True)
    d = jnp.exp(m2 - m1)
    p1 = 1.0 / (1.0 + d)
    p2 = 1.0 - p1

    h = jnp.dot(xb, w1_ref[...].astype(jnp.bfloat16),
                preferred_element_type=jnp.float32)
    ecol = jax.lax.broadcasted_iota(jnp.int32, (t, _E * _H), 1) // _H
    wfull = (jnp.where(ecol == i1, p1, 0.0)
             + jnp.where(ecol == i2, p2, 0.0))
    hw = (_silu(h) * wfull).astype(jnp.bfloat16)
    out = jnp.dot(hw, w2_ref[...].astype(jnp.bfloat16),
                  preferred_element_type=jnp.float32)

    a = jnp.dot(xb, s1_ref[...].astype(jnp.bfloat16),
                preferred_element_type=jnp.float32)
    b = jnp.dot(xb, s2_ref[...].astype(jnp.bfloat16),
                preferred_element_type=jnp.float32)
    out = out + jnp.dot((_silu(a) * b).astype(jnp.bfloat16),
                        s3_ref[...].astype(jnp.bfloat16),
                        preferred_element_type=jnp.float32)
    out_ref[...] = out


@jax.jit
def kernel(x, gate_w, fc1_w, fc2_w, sfc1, sfc2, sfc3):
    b, l, d = x.shape
    e, h, _ = fc1_w.shape
    s = sfc1.shape[0]
    n_tok = b * l
    xt = x.reshape(n_tok, d)

    gw_t = gate_w.T
    w1_t = fc1_w.reshape(e * h, d).T
    w2 = fc2_w.transpose(0, 2, 1).reshape(e * h, d)
    s1_t = sfc1.T
    s2_t = sfc2.T
    s3_t = sfc3.T

    tile = _TILE
    grid = (n_tok // tile,)

    def const(shape):
        return pl.BlockSpec(shape, lambda i: (0, 0))

    out = pl.pallas_call(
        _moe_body,
        grid=grid,
        in_specs=[
            pl.BlockSpec((tile, d), lambda i: (i, 0)),
            const(gw_t.shape), const(w1_t.shape), const(w2.shape),
            const(s1_t.shape), const(s2_t.shape), const(s3_t.shape),
        ],
        out_specs=pl.BlockSpec((tile, d), lambda i: (i, 0)),
        out_shape=jax.ShapeDtypeStruct((n_tok, d), jnp.float32),
    )(xt, gw_t, w1_t, w2, s1_t, s2_t, s3_t)
    return out.reshape(b, l, d)


# speedup vs baseline: 2.4857x; 1.0002x over previous
import functools

import jax
import jax.numpy as jnp
from jax.experimental import pallas as pl

_D = 768
_H = 256
_E = 8
_TILE = 1024


def _silu(v):
    return v * jax.nn.sigmoid(v)


def _moe_body(x_ref, gw_ref, w1_ref, w2_ref, s1_ref, s2_ref, s3_ref, out_ref):
    x = x_ref[...]
    xb = x.astype(jnp.bfloat16)
    t = x.shape[0]

    g = jnp.dot(x, gw_ref[...], preferred_element_type=jnp.float32)
    e_ids = jax.lax.broadcasted_iota(jnp.int32, g.shape, 1)
    m1 = jnp.max(g, axis=1, keepdims=True)
    i1 = jnp.min(jnp.where(g == m1, e_ids, _E), axis=1, keepdims=True)
    g2 = jnp.where(e_ids == i1, -jnp.inf, g)
    m2 = jnp.max(g2, axis=1, keepdims=True)
    i2 = jnp.min(jnp.where(g2 == m2, e_ids, _E), axis=1, keepdims=True)
    d = jnp.exp(m2 - m1)
    p1 = 1.0 / (1.0 + d)
    p2 = 1.0 - p1

    h = jnp.dot(xb, w1_ref[...].astype(jnp.bfloat16),
                preferred_element_type=jnp.float32)
    ecol = jax.lax.broadcasted_iota(jnp.int32, (t, _E * _H), 1) // _H
    wfull = (jnp.where(ecol == i1, p1, 0.0)
             + jnp.where(ecol == i2, p2, 0.0))
    hw = (_silu(h) * wfull).astype(jnp.bfloat16)
    out = jnp.dot(hw, w2_ref[...].astype(jnp.bfloat16),
                  preferred_element_type=jnp.float32)

    a = jnp.dot(xb, s1_ref[...].astype(jnp.bfloat16),
                preferred_element_type=jnp.float32)
    b = jnp.dot(xb, s2_ref[...].astype(jnp.bfloat16),
                preferred_element_type=jnp.float32)
    out = out + jnp.dot((_silu(a) * b).astype(jnp.bfloat16),
                        s3_ref[...].astype(jnp.bfloat16),
                        preferred_element_type=jnp.float32)
    out_ref[...] = out


@jax.jit
def kernel(x, gate_w, fc1_w, fc2_w, sfc1, sfc2, sfc3):
    b, l, d = x.shape
    e, h, _ = fc1_w.shape
    s = sfc1.shape[0]
    n_tok = b * l
    xt = x.reshape(n_tok, d)

    gw_t = gate_w.T
    w1_t = fc1_w.reshape(e * h, d).T
    w2 = fc2_w.transpose(0, 2, 1).reshape(e * h, d)
    s1_t = sfc1.T
    s2_t = sfc2.T
    s3_t = sfc3.T

    tile = _TILE
    grid = (n_tok // tile,)

    def const(shape):
        return pl.BlockSpec(shape, lambda i: (0, 0))

    out = pl.pallas_call(
        _moe_body,
        grid=grid,
        in_specs=[
            pl.BlockSpec((tile, d), lambda i: (i, 0)),
            const(gw_t.shape), const(w1_t.shape), const(w2.shape),
            const(s1_t.shape), const(s2_t.shape), const(s3_t.shape),
        ],
        out_specs=pl.BlockSpec((tile, d), lambda i: (i, 0)),
        out_shape=jax.ShapeDtypeStruct((n_tok, d), jnp.float32),
    )(xt, gw_t, w1_t, w2, s1_t, s2_t, s3_t)
    return out.reshape(b, l, d)
